# baseline (speedup 1.0000x reference)
import jax
import jax.numpy as jnp
from jax import lax
from jax.experimental import pallas as pl
from jax.experimental.pallas import tpu as pltpu
from jax.experimental.pallas import tpu_sc as plsc

N = 10000
E = 320000
D = 128
H = 128
O = 64

NC = 2
NS = 16
NW = NC * NS
K = 100
KB = 2
NBLK = 50
EPW = NBLK * KB * K
N_PAD = 10240
RPS = N_PAD // NS
ZB = 16

_mesh = plsc.VectorSubcoreMesh(core_axis_name="c", subcore_axis_name="s")


def _zero_acc(zf, acc, rbase, zsem):
    zero16 = jnp.zeros((16,), dtype=jnp.float32)
    for i in range(ZB):
        for j in range(8):
            zf[i, pl.ds(j * 16, 16)] = zero16

    def zloop(i, _):
        pltpu.async_copy(zf, acc.at[pl.ds(rbase + i * ZB, ZB)], zsem)
        return 0
    lax.fori_loop(0, RPS // ZB, zloop, 0)

    def zdrain(i, _):
        pltpu.make_async_copy(zf, acc.at[pl.ds(rbase + i * ZB, ZB)],
                              zsem).wait()
        return 0
    lax.fori_loop(0, RPS // ZB, zdrain, 0)


def _seg_body(h_hbm, eidx_hbm, out_hbm, ib0, ib1, rows0, rows1, zf, acc,
              is0, is1, gs0, gs1, zsem):
    cid = lax.axis_index("c")
    sid = lax.axis_index("s")
    wid = sid * NC + cid
    rbase = sid * RPS

    ibs = (ib0, ib1)
    isems = (is0, is1)
    rbufs = (rows0, rows1)
    gsems = (gs0, gs1)
    base = wid * NBLK

    def fetch_block(b, p):
        pltpu.async_copy(eidx_hbm.at[base + b], ibs[p], isems[p])

    def fetch_wait(b, p):
        pltpu.make_async_copy(eidx_hbm.at[base + b], ibs[p], isems[p]).wait()

    def gstart(p, j, rp):
        pltpu.async_copy(h_hbm.at[ibs[p].at[j]], rbufs[rp], gsems[rp])

    def gwait(p, j, rp):
        pltpu.make_async_copy(h_hbm.at[ibs[p].at[j]], rbufs[rp],
                              gsems[rp]).wait()

    def scat(p, j, rp):
        pltpu.sync_copy(rbufs[rp], acc.at[ibs[p].at[KB + j]], add=True)

    fetch_block(0, 0)
    fetch_wait(0, 0)
    gstart(0, 0, 0)
    _zero_acc(zf, acc, rbase, zsem)
    plsc.subcore_barrier()

    def do_block(b, p):
        @pl.when(b < NBLK - 1)
        def _():
            fetch_block(b + 1, p ^ 1)
        gstart(p, 1, 1)
        gwait(p, 0, 0)
        scat(p, 0, 0)

        @pl.when(b < NBLK - 1)
        def _():
            fetch_wait(b + 1, p ^ 1)
            gstart(p ^ 1, 0, 0)
        gwait(p, 1, 1)
        scat(p, 1, 1)

    def body(b2, _):
        do_block(b2 * 2, 0)
        do_block(b2 * 2 + 1, 1)
        return 0
    lax.fori_loop(0, NBLK // 2, body, 0)

    plsc.subcore_barrier()

    pltpu.sync_copy(acc.at[pl.ds(rbase, RPS)],
                    out_hbm.at[pl.ds(cid * N_PAD + rbase, RPS)])


_seg = pl.kernel(
    _seg_body,
    out_type=jax.ShapeDtypeStruct((NC * N_PAD, 128), jnp.float32),
    mesh=_mesh,
    scratch_types=[
        pltpu.VMEM((2 * KB, K), jnp.int32),
        pltpu.VMEM((2 * KB, K), jnp.int32),
        pltpu.VMEM((K, 128), jnp.float32),
        pltpu.VMEM((K, 128), jnp.float32),
        pltpu.VMEM((ZB, 128), jnp.float32),
        pltpu.VMEM_SHARED((N_PAD, 128), jnp.float32),
        pltpu.SemaphoreType.DMA,
        pltpu.SemaphoreType.DMA,
        pltpu.SemaphoreType.DMA,
        pltpu.SemaphoreType.DMA,
        pltpu.SemaphoreType.DMA,
    ],
)


HB = 640


def _deg_body(eidx_hbm, out_hbm, ib0, ib1, hist, is0, is1):
    cid = lax.axis_index("c")
    sid = lax.axis_index("s")
    wid = sid * NC + cid
    base = wid * NBLK

    zero16 = jnp.zeros((16,), dtype=jnp.float32)
    one16 = jnp.full((16,), 1.0, dtype=jnp.float32)
    for i in range(HB):
        hist[i] = zero16

    ibs = (ib0, ib1)
    isems = (is0, is1)

    def fetch_block(b, p):
        pltpu.async_copy(eidx_hbm.at[base + b], ibs[p], isems[p])

    def fetch_wait(b, p):
        pltpu.make_async_copy(eidx_hbm.at[base + b], ibs[p], isems[p]).wait()

    fetch_block(0, 0)
    tailmask = lax.iota(jnp.int32, 16) >= 12

    def hist_add(dst, mask=None):
        plsc.addupdate_scatter(
            hist, [lax.shift_right_logical(dst, 4), dst & 15], one16,
            mask=mask)

    def do_block(b, p):
        @pl.when(b < NBLK - 1)
        def _():
            fetch_block(b + 1, p ^ 1)
        fetch_wait(b, p)
        for j in range(KB):
            for o in range(0, 96, 16):
                hist_add(ibs[p][KB + j, pl.ds(o, 16)])
            hist_add(ibs[p][KB + j, pl.ds(84, 16)], tailmask)

    def body(b2, _):
        do_block(b2 * 2, 0)
        do_block(b2 * 2 + 1, 1)
        return 0
    lax.fori_loop(0, NBLK // 2, body, 0)

    pltpu.sync_copy(hist, out_hbm.at[wid])


_deg_kernel = pl.kernel(
    _deg_body,
    out_type=jax.ShapeDtypeStruct((NW, HB, 16), jnp.float32),
    mesh=_mesh,
    compiler_params=pltpu.CompilerParams(needs_layout_passes=False),
    scratch_types=[
        pltpu.VMEM((2 * KB, K), jnp.int32),
        pltpu.VMEM((2 * KB, K), jnp.int32),
        pltpu.VMEM((HB, 16), jnp.float32),
        pltpu.SemaphoreType.DMA,
        pltpu.SemaphoreType.DMA,
    ],
)


BN = 1000
GRID = N // BN


def _mean(a_ref, recip):
    return (a_ref[0] + a_ref[1]) * recip


def _tc1_body(a_ref, degh_ref, x_ref, wl_ref, wr_ref, b_ref, o_ref, r_ref):
    deg = jnp.sum(degh_ref[...], axis=0)
    recip = 1.0 / jnp.maximum(deg, 1.0)
    r_ref[...] = recip
    mean = _mean(a_ref, recip)
    h = (jnp.dot(mean, wl_ref[...], preferred_element_type=jnp.float32)
         + jnp.dot(x_ref[...], wr_ref[...], preferred_element_type=jnp.float32)
         + b_ref[...])
    o_ref[...] = jnp.maximum(h, 0.0)


def _tc2_body(a_ref, r_ref, h1_ref, x_ref, wl_ref, wr_ref, b_ref,
              wres_ref, bres_ref, o_ref):
    mean = _mean(a_ref, r_ref[...])
    h = (jnp.dot(mean, wl_ref[...], preferred_element_type=jnp.float32)
         + jnp.dot(h1_ref[...], wr_ref[...], preferred_element_type=jnp.float32)
         + b_ref[...])
    o_ref[...] = (jnp.maximum(h, 0.0)
                  + jnp.dot(x_ref[...], wres_ref[...],
                            preferred_element_type=jnp.float32)
                  + bres_ref[...])


def _tc3_body(a_ref, r_ref, h_ref, wmul_ref, wmur_ref, bmu_ref,
              wlsl_ref, wlsr_ref, bls_ref, mu_ref, ls_ref):
    mean = _mean(a_ref, r_ref[...])
    mu_ref[...] = (jnp.dot(mean, wmul_ref[...], preferred_element_type=jnp.float32)
                   + jnp.dot(h_ref[...], wmur_ref[...],
                             preferred_element_type=jnp.float32)
                   + bmu_ref[...])
    ls_ref[...] = (jnp.dot(mean, wlsl_ref[...], preferred_element_type=jnp.float32)
                   + jnp.dot(h_ref[...], wlsr_ref[...],
                             preferred_element_type=jnp.float32)
                   + bls_ref[...])


def _a_spec():
    return pl.BlockSpec((NC, BN, 128), lambda i: (0, i, 0))


def _degh_spec():
    return pl.BlockSpec((NW, BN, 1), lambda i: (0, i, 0))


def _r_spec():
    return pl.BlockSpec((BN, 1), lambda i: (i, 0))


def _row_spec(width):
    return pl.BlockSpec((BN, width), lambda i: (i, 0))


def _w_spec(r, c):
    return pl.BlockSpec((r, c), lambda i: (0, 0))


def _tc1(a, degh, x, wl, wr, b):
    return pl.pallas_call(
        _tc1_body,
        out_shape=(jax.ShapeDtypeStruct((N, H), jnp.float32),
                   jax.ShapeDtypeStruct((N, 1), jnp.float32)),
        grid=(GRID,),
        in_specs=[_a_spec(), _degh_spec(), _row_spec(D),
                  _w_spec(D, H), _w_spec(D, H), _w_spec(1, H)],
        out_specs=(_row_spec(H), _r_spec()),
    )(a, degh, x, wl, wr, b)


def _tc2(a, r, h1, x, wl, wr, b, wres, bres):
    return pl.pallas_call(
        _tc2_body,
        out_shape=jax.ShapeDtypeStruct((N, H), jnp.float32),
        grid=(GRID,),
        in_specs=[_a_spec(), _r_spec(), _row_spec(H), _row_spec(D),
                  _w_spec(H, H), _w_spec(H, H), _w_spec(1, H),
                  _w_spec(D, H), _w_spec(1, H)],
        out_specs=_row_spec(H),
    )(a, r, h1, x, wl, wr, b, wres, bres)


def _tc3(a, r, h, wmul, wmur, bmu, wlsl, wlsr, bls):
    return pl.pallas_call(
        _tc3_body,
        out_shape=(jax.ShapeDtypeStruct((N, O), jnp.float32),
                   jax.ShapeDtypeStruct((N, O), jnp.float32)),
        grid=(GRID,),
        in_specs=[_a_spec(), _r_spec(), _row_spec(H),
                  _w_spec(H, O), _w_spec(H, O), _w_spec(1, O),
                  _w_spec(H, O), _w_spec(H, O), _w_spec(1, O)],
        out_specs=(_row_spec(O), _row_spec(O)),
    )(a, r, h, wmul, wmur, bmu, wlsl, wlsr, bls)


def kernel(x, edge_index, W1l, W1r, b1, W2l, W2r, b2, Wres, bres,
           Wmul, Wmur, bmu, Wlsl, Wlsr, bls):
    src = edge_index[0].astype(jnp.int32)
    dst = edge_index[1].astype(jnp.int32)
    src4 = src.reshape(NW, NBLK, KB, K)
    dst4 = dst.reshape(NW, NBLK, KB, K)
    eidx = jnp.concatenate([src4, dst4], axis=2).reshape(NW * NBLK, 2 * KB, K)

    degh = _deg_kernel(eidx).reshape(NW, HB * 16, 1)
    a1 = _seg(x, eidx).reshape(NC, N_PAD, 128)
    h1, recip = _tc1(a1, degh, x, W1l, W1r, b1.reshape(1, H))
    a2 = _seg(h1, eidx).reshape(NC, N_PAD, 128)
    h = _tc2(a2, recip, h1, x, W2l, W2r, b2.reshape(1, H),
             Wres, bres.reshape(1, H))
    a3 = _seg(h, eidx).reshape(NC, N_PAD, 128)
    mu, logstd = _tc3(a3, recip, h, Wmul, Wmur, bmu.reshape(1, O),
                      Wlsl, Wlsr, bls.reshape(1, O))
    return (mu, logstd)


# speedup vs baseline: 1.0151x; 1.0151x over previous
import jax
import jax.numpy as jnp
from jax import lax
from jax.experimental import pallas as pl
from jax.experimental.pallas import tpu as pltpu
from jax.experimental.pallas import tpu_sc as plsc

N = 10000
E = 320000
D = 128
H = 128
O = 64

NC = 2
NS = 16
NW = NC * NS
K = 100
KB = 2
NBLK = 50
EPW = NBLK * KB * K
N_PAD = 10240
RPS = N_PAD // NS
ZB = 16

_mesh = plsc.VectorSubcoreMesh(core_axis_name="c", subcore_axis_name="s")


def _zero_acc(zf, acc, rbase, zsem):
    zero16 = jnp.zeros((16,), dtype=jnp.float32)
    for i in range(ZB):
        for j in range(8):
            zf[i, pl.ds(j * 16, 16)] = zero16

    def zloop(i, _):
        pltpu.async_copy(zf, acc.at[pl.ds(rbase + i * ZB, ZB)], zsem)
        return 0
    lax.fori_loop(0, RPS // ZB, zloop, 0)

    def zdrain(i, _):
        pltpu.make_async_copy(zf, acc.at[pl.ds(rbase + i * ZB, ZB)],
                              zsem).wait()
        return 0
    lax.fori_loop(0, RPS // ZB, zdrain, 0)


def _seg_body(h_hbm, eidx_hbm, out_hbm, ib0, ib1, rows0, rows1, zf, acc,
              is0, is1, gs0, gs1, zsem):
    cid = lax.axis_index("c")
    sid = lax.axis_index("s")
    wid = sid * NC + cid
    rbase = sid * RPS

    ibs = (ib0, ib1)
    isems = (is0, is1)
    rbufs = (rows0, rows1)
    gsems = (gs0, gs1)
    base = wid * NBLK

    def fetch_block(b, p):
        pltpu.async_copy(eidx_hbm.at[base + b], ibs[p], isems[p])

    def fetch_wait(b, p):
        pltpu.make_async_copy(eidx_hbm.at[base + b], ibs[p], isems[p]).wait()

    def gstart(p, j, rp):
        pltpu.async_copy(h_hbm.at[ibs[p].at[j]], rbufs[rp], gsems[rp])

    def gwait(p, j, rp):
        pltpu.make_async_copy(h_hbm.at[ibs[p].at[j]], rbufs[rp],
                              gsems[rp]).wait()

    def scat(p, j, rp):
        pltpu.sync_copy(rbufs[rp], acc.at[ibs[p].at[KB + j]], add=True)

    fetch_block(0, 0)
    fetch_wait(0, 0)
    gstart(0, 0, 0)
    _zero_acc(zf, acc, rbase, zsem)
    plsc.subcore_barrier()

    def do_block(b, p):
        @pl.when(b < NBLK - 1)
        def _():
            fetch_block(b + 1, p ^ 1)
        gstart(p, 1, 1)
        gwait(p, 0, 0)
        scat(p, 0, 0)

        @pl.when(b < NBLK - 1)
        def _():
            fetch_wait(b + 1, p ^ 1)
            gstart(p ^ 1, 0, 0)
        gwait(p, 1, 1)
        scat(p, 1, 1)

    def body(b2, _):
        do_block(b2 * 2, 0)
        do_block(b2 * 2 + 1, 1)
        return 0
    lax.fori_loop(0, NBLK // 2, body, 0)

    plsc.subcore_barrier()

    pltpu.sync_copy(acc.at[pl.ds(rbase, RPS)],
                    out_hbm.at[pl.ds(cid * N_PAD + rbase, RPS)])


_seg = pl.kernel(
    _seg_body,
    out_type=jax.ShapeDtypeStruct((NC * N_PAD, 128), jnp.float32),
    mesh=_mesh,
    scratch_types=[
        pltpu.VMEM((2 * KB, K), jnp.int32),
        pltpu.VMEM((2 * KB, K), jnp.int32),
        pltpu.VMEM((K, 128), jnp.float32),
        pltpu.VMEM((K, 128), jnp.float32),
        pltpu.VMEM((ZB, 128), jnp.float32),
        pltpu.VMEM_SHARED((N_PAD, 128), jnp.float32),
        pltpu.SemaphoreType.DMA,
        pltpu.SemaphoreType.DMA,
        pltpu.SemaphoreType.DMA,
        pltpu.SemaphoreType.DMA,
        pltpu.SemaphoreType.DMA,
    ],
)


HB = 640


GB = 10
NG = NBLK // GB


def _deg_body(eidx_hbm, out_hbm, bb0, bb1, hist, is0, is1):
    cid = lax.axis_index("c")
    sid = lax.axis_index("s")
    wid = sid * NC + cid
    base = wid * NBLK

    zero16 = jnp.zeros((16,), dtype=jnp.float32)
    one16 = jnp.full((16,), 1.0, dtype=jnp.float32)

    bbs = (bb0, bb1)
    isems = (is0, is1)

    def fetch_group(g, p):
        pltpu.async_copy(eidx_hbm.at[pl.ds(base + g * GB, GB)], bbs[p],
                         isems[p])

    def group_wait(g, p):
        pltpu.make_async_copy(eidx_hbm.at[pl.ds(base + g * GB, GB)], bbs[p],
                              isems[p]).wait()

    fetch_group(0, 0)
    for i in range(HB):
        hist[i] = zero16

    tailmask = lax.iota(jnp.int32, 16) >= 12

    def hist_add(dst, mask=None):
        plsc.addupdate_scatter(
            hist, [lax.shift_right_logical(dst, 4), dst & 15], one16,
            mask=mask)

    for g in range(NG):
        p = g % 2
        if g + 1 < NG:
            fetch_group(g + 1, p ^ 1)
        group_wait(g, p)
        for t in range(GB):
            for j in range(KB):
                for o in range(0, 96, 16):
                    hist_add(bbs[p][t, KB + j, pl.ds(o, 16)])
                hist_add(bbs[p][t, KB + j, pl.ds(84, 16)], tailmask)

    pltpu.sync_copy(hist, out_hbm.at[wid])


_deg_kernel = pl.kernel(
    _deg_body,
    out_type=jax.ShapeDtypeStruct((NW, HB, 16), jnp.float32),
    mesh=_mesh,
    compiler_params=pltpu.CompilerParams(needs_layout_passes=False),
    scratch_types=[
        pltpu.VMEM((GB, 2 * KB, K), jnp.int32),
        pltpu.VMEM((GB, 2 * KB, K), jnp.int32),
        pltpu.VMEM((HB, 16), jnp.float32),
        pltpu.SemaphoreType.DMA,
        pltpu.SemaphoreType.DMA,
    ],
)


BN = 1000
GRID = N // BN


def _mean(a_ref, recip):
    return (a_ref[0] + a_ref[1]) * recip


def _tc1_body(a_ref, degh_ref, x_ref, wl_ref, wr_ref, b_ref, o_ref, r_ref):
    deg = jnp.sum(degh_ref[...], axis=0)
    recip = 1.0 / jnp.maximum(deg, 1.0)
    r_ref[...] = recip
    mean = _mean(a_ref, recip)
    h = (jnp.dot(mean, wl_ref[...], preferred_element_type=jnp.float32)
         + jnp.dot(x_ref[...], wr_ref[...], preferred_element_type=jnp.float32)
         + b_ref[...])
    o_ref[...] = jnp.maximum(h, 0.0)


def _tc2_body(a_ref, r_ref, h1_ref, x_ref, wl_ref, wr_ref, b_ref,
              wres_ref, bres_ref, o_ref):
    mean = _mean(a_ref, r_ref[...])
    h = (jnp.dot(mean, wl_ref[...], preferred_element_type=jnp.float32)
         + jnp.dot(h1_ref[...], wr_ref[...], preferred_element_type=jnp.float32)
         + b_ref[...])
    o_ref[...] = (jnp.maximum(h, 0.0)
                  + jnp.dot(x_ref[...], wres_ref[...],
                            preferred_element_type=jnp.float32)
                  + bres_ref[...])


def _tc3_body(a_ref, r_ref, h_ref, wmul_ref, wmur_ref, bmu_ref,
              wlsl_ref, wlsr_ref, bls_ref, mu_ref, ls_ref):
    mean = _mean(a_ref, r_ref[...])
    mu_ref[...] = (jnp.dot(mean, wmul_ref[...], preferred_element_type=jnp.float32)
                   + jnp.dot(h_ref[...], wmur_ref[...],
                             preferred_element_type=jnp.float32)
                   + bmu_ref[...])
    ls_ref[...] = (jnp.dot(mean, wlsl_ref[...], preferred_element_type=jnp.float32)
                   + jnp.dot(h_ref[...], wlsr_ref[...],
                             preferred_element_type=jnp.float32)
                   + bls_ref[...])


def _a_spec():
    return pl.BlockSpec((NC, BN, 128), lambda i: (0, i, 0))


def _degh_spec():
    return pl.BlockSpec((NW, BN, 1), lambda i: (0, i, 0))


def _r_spec():
    return pl.BlockSpec((BN, 1), lambda i: (i, 0))


def _row_spec(width):
    return pl.BlockSpec((BN, width), lambda i: (i, 0))


def _w_spec(r, c):
    return pl.BlockSpec((r, c), lambda i: (0, 0))


def _tc1(a, degh, x, wl, wr, b):
    return pl.pallas_call(
        _tc1_body,
        out_shape=(jax.ShapeDtypeStruct((N, H), jnp.float32),
                   jax.ShapeDtypeStruct((N, 1), jnp.float32)),
        grid=(GRID,),
        in_specs=[_a_spec(), _degh_spec(), _row_spec(D),
                  _w_spec(D, H), _w_spec(D, H), _w_spec(1, H)],
        out_specs=(_row_spec(H), _r_spec()),
    )(a, degh, x, wl, wr, b)


def _tc2(a, r, h1, x, wl, wr, b, wres, bres):
    return pl.pallas_call(
        _tc2_body,
        out_shape=jax.ShapeDtypeStruct((N, H), jnp.float32),
        grid=(GRID,),
        in_specs=[_a_spec(), _r_spec(), _row_spec(H), _row_spec(D),
                  _w_spec(H, H), _w_spec(H, H), _w_spec(1, H),
                  _w_spec(D, H), _w_spec(1, H)],
        out_specs=_row_spec(H),
    )(a, r, h1, x, wl, wr, b, wres, bres)


def _tc3(a, r, h, wmul, wmur, bmu, wlsl, wlsr, bls):
    return pl.pallas_call(
        _tc3_body,
        out_shape=(jax.ShapeDtypeStruct((N, O), jnp.float32),
                   jax.ShapeDtypeStruct((N, O), jnp.float32)),
        grid=(GRID,),
        in_specs=[_a_spec(), _r_spec(), _row_spec(H),
                  _w_spec(H, O), _w_spec(H, O), _w_spec(1, O),
                  _w_spec(H, O), _w_spec(H, O), _w_spec(1, O)],
        out_specs=(_row_spec(O), _row_spec(O)),
    )(a, r, h, wmul, wmur, bmu, wlsl, wlsr, bls)


def kernel(x, edge_index, W1l, W1r, b1, W2l, W2r, b2, Wres, bres,
           Wmul, Wmur, bmu, Wlsl, Wlsr, bls):
    src = edge_index[0].astype(jnp.int32)
    dst = edge_index[1].astype(jnp.int32)
    src4 = src.reshape(NW, NBLK, KB, K)
    dst4 = dst.reshape(NW, NBLK, KB, K)
    eidx = jnp.concatenate([src4, dst4], axis=2).reshape(NW * NBLK, 2 * KB, K)

    degh = _deg_kernel(eidx).reshape(NW, HB * 16, 1)
    a1 = _seg(x, eidx).reshape(NC, N_PAD, 128)
    h1, recip = _tc1(a1, degh, x, W1l, W1r, b1.reshape(1, H))
    a2 = _seg(h1, eidx).reshape(NC, N_PAD, 128)
    h = _tc2(a2, recip, h1, x, W2l, W2r, b2.reshape(1, H),
             Wres, bres.reshape(1, H))
    a3 = _seg(h, eidx).reshape(NC, N_PAD, 128)
    mu, logstd = _tc3(a3, recip, h, Wmul, Wmur, bmu.reshape(1, O),
                      Wlsl, Wlsr, bls.reshape(1, O))
    return (mu, logstd)
